# Initial kernel scaffold; baseline (speedup 1.0000x reference)
import jax
import jax.numpy as jnp
from jax.experimental import pallas as pl


def kernel(expert_nodes, expert_links, graph_pool, expert_candidate, mask_expert, params):
    raise NotImplementedError("write your pallas kernel here")


# speedup vs baseline: 1.5542x; 1.5542x over previous
import jax
import jax.numpy as jnp
from jax import lax
from jax.experimental import pallas as pl

B, NE, D = 128, 64, 1024
R = B * NE
BB = 16
RB = BB * NE
GRID = B // BB
CD = 256
EPS = 1e-5


def _bnrelu(z, stats, gamma, beta):
    x = gamma * (z - stats[0:1, :]) / jnp.sqrt(stats[1:2, :] + EPS) + beta
    return jnp.maximum(x, 0.0)


def _pool_mm(x, links, w, b, z_ref):
    x3 = x.reshape(BB, NE, D)
    pooled = lax.dot_general(
        links, x3, (((2,), (1,)), ((0,), (0,))),
        preferred_element_type=jnp.float32) + x3
    z_ref[...] = jnp.dot(pooled.reshape(RB, D), w,
                         preferred_element_type=jnp.float32) + b


def _entry_kernel(x_ref, links_ref, w_ref, b_ref, z_ref):
    _pool_mm(x_ref[...], links_ref[...], w_ref[...], b_ref[...], z_ref)


def _entry_bn_kernel(x_ref, sin_ref, g_ref, be_ref, links_ref, w_ref, b_ref,
                     z_ref):
    x = _bnrelu(x_ref[...], sin_ref[...], g_ref[...], be_ref[...])
    _pool_mm(x, links_ref[...], w_ref[...], b_ref[...], z_ref)


def _mid_kernel(x_ref, sin_ref, g_ref, be_ref, w_ref, b_ref, z_ref):
    x = _bnrelu(x_ref[...], sin_ref[...], g_ref[...], be_ref[...])
    z_ref[...] = jnp.dot(x, w_ref[...],
                         preferred_element_type=jnp.float32) + b_ref[...]


def _stats_kernel(z_ref, o_ref):
    z = z_ref[...]
    m = jnp.mean(z, axis=0, keepdims=True)
    c = z - m
    o_ref[0:1, :] = m
    o_ref[1:2, :] = jnp.mean(c * c, axis=0, keepdims=True)


def _final_kernel(x_ref, sin_ref, g_ref, be_ref, pool_ref, cand_ref, links_ref,
                  w0_ref, b0_ref, w1_ref, b1_ref, w2_ref, b2_ref,
                  ei_ref, ef_ref, el_ref, le_ref):
    h = _bnrelu(x_ref[...], sin_ref[...], g_ref[...], be_ref[...])
    h3 = h.reshape(BB, NE, D)
    pv = lax.dot_general(
        pool_ref[...].reshape(BB, 1, NE), h3, (((2,), (1,)), ((0,), (0,))),
        preferred_element_type=jnp.float32).reshape(BB, D)
    cand = cand_ref[...].astype(jnp.int32)
    col = lax.broadcasted_iota(jnp.int32, (BB, NE, NE), 2)
    onehot = (cand[:, :, None] == col).astype(jnp.float32)
    cf = lax.dot_general(
        onehot, h3, (((2,), (1,)), ((0,), (0,))),
        preferred_element_type=jnp.float32)
    pvb = jnp.broadcast_to(pv[:, None, :], (BB, NE, D))
    ctx = jnp.concatenate([cf, pvb], axis=-1).reshape(RB, 2 * D)
    u = jnp.dot(ctx, w0_ref[...], preferred_element_type=jnp.float32) + b0_ref[...]
    a1 = jnp.tanh(u)
    a2 = jnp.tanh(jnp.dot(a1, w1_ref[...],
                          preferred_element_type=jnp.float32) + b1_ref[...])
    s = (jnp.dot(a2, w2_ref[...], preferred_element_type=jnp.float32)
         + b2_ref[...]).reshape(BB, NE)
    rowmax = jnp.max(s, axis=1, keepdims=True)
    e = jnp.exp(s - rowmax)
    denom = jnp.sum(e, axis=1, keepdims=True)
    probs = e / denom
    action = jnp.max(probs, axis=1, keepdims=True)
    nid = lax.broadcasted_iota(jnp.int32, (BB, NE), 1)
    sel = jnp.min(jnp.where(probs == action, nid, NE), axis=1, keepdims=True)
    selmask = (nid == sel).astype(jnp.float32)
    ei = jnp.sum(selmask * cand.astype(jnp.float32), axis=1,
                 keepdims=True).astype(jnp.int32)
    featmask = (nid == ei).astype(jnp.float32)
    ef = jnp.sum(h3 * featmask[:, :, None], axis=1)
    el = jnp.sum(links_ref[...] * selmask[:, :, None], axis=1)
    ei_ref[...] = ei.astype(ei_ref.dtype)
    ef_ref[...] = ef
    el_ref[...] = el
    le_ref[...] = jnp.log(action + 1e-12)


def _row_spec():
    return pl.BlockSpec((RB, D), lambda i: (i, 0))


def _const_spec(shape):
    return pl.BlockSpec(shape, lambda i: tuple(0 for _ in shape))


def _batch_spec(shape):
    return pl.BlockSpec(shape, lambda i: (i,) + tuple(0 for _ in shape[1:]))


def _stats(z):
    return pl.pallas_call(
        _stats_kernel,
        grid=(D // CD,),
        in_specs=[pl.BlockSpec((R, CD), lambda i: (0, i))],
        out_specs=pl.BlockSpec((2, CD), lambda i: (0, i)),
        out_shape=jax.ShapeDtypeStruct((2, D), jnp.float32),
    )(z)


def kernel(expert_nodes, expert_links, graph_pool, expert_candidate,
           mask_expert, params):
    del mask_expert
    x0 = expert_nodes.reshape(R, D)
    zs, zspec = jax.ShapeDtypeStruct((R, D), jnp.float32), _row_spec()
    links_spec = _batch_spec((BB, NE, NE))

    z = pl.pallas_call(
        _entry_kernel,
        grid=(GRID,),
        in_specs=[_row_spec(), links_spec, _const_spec((D, D)),
                  _const_spec((1, D))],
        out_specs=zspec,
        out_shape=zs,
    )(x0, expert_links, params['gin0_w1'], params['gin0_b1'].reshape(1, D))
    s = _stats(z)

    def mid(z, s, l):
        return pl.pallas_call(
            _mid_kernel,
            grid=(GRID,),
            in_specs=[_row_spec(), _const_spec((2, D)), _const_spec((1, D)),
                      _const_spec((1, D)), _const_spec((D, D)),
                      _const_spec((1, D))],
            out_specs=zspec,
            out_shape=zs,
        )(z, s, params['gin%d_bn1_g' % l].reshape(1, D),
          params['gin%d_bn1_b' % l].reshape(1, D), params['gin%d_w2' % l],
          params['gin%d_b2' % l].reshape(1, D))

    z = mid(z, s, 0)
    s = _stats(z)

    z = pl.pallas_call(
        _entry_bn_kernel,
        grid=(GRID,),
        in_specs=[_row_spec(), _const_spec((2, D)), _const_spec((1, D)),
                  _const_spec((1, D)), links_spec, _const_spec((D, D)),
                  _const_spec((1, D))],
        out_specs=zspec,
        out_shape=zs,
    )(z, s, params['gin0_bn2_g'].reshape(1, D),
      params['gin0_bn2_b'].reshape(1, D), expert_links, params['gin1_w1'],
      params['gin1_b1'].reshape(1, D))
    s = _stats(z)

    z = mid(z, s, 1)
    s = _stats(z)

    idt = expert_candidate.dtype
    ei, ef, el, le = pl.pallas_call(
        _final_kernel,
        grid=(GRID,),
        in_specs=[_row_spec(), _const_spec((2, D)), _const_spec((1, D)),
                  _const_spec((1, D)), _batch_spec((BB, NE)),
                  _batch_spec((BB, NE)), links_spec,
                  _const_spec((2 * D, D)), _const_spec((1, D)),
                  _const_spec((D, D)), _const_spec((1, D)),
                  _const_spec((D, 1)), _const_spec((1, 1))],
        out_specs=[pl.BlockSpec((BB, 1), lambda i: (i, 0)),
                   pl.BlockSpec((BB, D), lambda i: (i, 0)),
                   pl.BlockSpec((BB, NE), lambda i: (i, 0)),
                   pl.BlockSpec((BB, 1), lambda i: (i, 0))],
        out_shape=[jax.ShapeDtypeStruct((B, 1), idt),
                   jax.ShapeDtypeStruct((B, D), jnp.float32),
                   jax.ShapeDtypeStruct((B, NE), jnp.float32),
                   jax.ShapeDtypeStruct((B, 1), jnp.float32)],
    )(z, s, params['gin1_bn2_g'].reshape(1, D),
      params['gin1_bn2_b'].reshape(1, D), graph_pool, expert_candidate,
      expert_links, params['act0_w'], params['act0_b'].reshape(1, D),
      params['act1_w'], params['act1_b'].reshape(1, D), params['act2_w'],
      params['act2_b'].reshape(1, 1))

    return ei[:, 0], ef, el, le[:, 0]
